# Initial kernel scaffold; baseline (speedup 1.0000x reference)
import jax
import jax.numpy as jnp
from jax.experimental import pallas as pl


def kernel(node_attr, edge_index, edge_attr, edge_sh, node_mlp, edge_mlp, layers):
    raise NotImplementedError("write your pallas kernel here")


# speedup vs baseline: 26.8864x; 26.8864x over previous
import functools

import jax
import jax.numpy as jnp
from jax.experimental import pallas as pl
from jax.experimental.pallas import tpu as pltpu

N_NODES = 10000
N_EDGES = 160000
NS = 32

F32 = jnp.float32


def _dot(a, b):
    return jnp.dot(a, b, preferred_element_type=F32)


def _mlp2_body(x_ref, w1_ref, b1_ref, w2_ref, b2_ref, o_ref):
    t = jnp.maximum(_dot(x_ref[...], w1_ref[...]) + b1_ref[...], 0.0)
    o_ref[...] = _dot(t, w2_ref[...]) + b2_ref[...]


def _mlp2_rows(x, params, block_rows):
    W1, b1, W2, b2 = params
    n, din = x.shape
    dmid, dout = W1.shape[1], W2.shape[1]
    grid = (n // block_rows,)
    return pl.pallas_call(
        _mlp2_body,
        grid=grid,
        in_specs=[
            pl.BlockSpec((block_rows, din), lambda i: (i, 0)),
            pl.BlockSpec((din, dmid), lambda i: (0, 0)),
            pl.BlockSpec((1, dmid), lambda i: (0, 0)),
            pl.BlockSpec((dmid, dout), lambda i: (0, 0)),
            pl.BlockSpec((1, dout), lambda i: (0, 0)),
        ],
        out_specs=pl.BlockSpec((block_rows, dout), lambda i: (i, 0)),
        out_shape=jax.ShapeDtypeStruct((n, dout), F32),
    )(x, W1, b1.reshape(1, -1), W2, b2.reshape(1, -1))


def _matmul_body(x_ref, w_ref, o_ref):
    o_ref[...] = _dot(x_ref[...], w_ref[...])


def _node_linear(x, w_pad, block_rows):
    n, din = x.shape
    dp = w_pad.shape[1]
    return pl.pallas_call(
        _matmul_body,
        grid=(n // block_rows,),
        in_specs=[
            pl.BlockSpec((block_rows, din), lambda i: (i, 0)),
            pl.BlockSpec((din, dp), lambda i: (0, 0)),
        ],
        out_specs=pl.BlockSpec((block_rows, dp), lambda i: (i, 0)),
        out_shape=jax.ShapeDtypeStruct((n, dp), F32),
    )(x, w_pad)


def _edge_body(ea_ref, xs_ref, xd_ref, sh_ref, hs_ref,
               w1a_ref, w1b_ref, w1c_ref, b1_ref, w2_ref, b2_ref, wsh_ref,
               o_ref):
    t = (_dot(ea_ref[...], w1a_ref[...]) + _dot(xs_ref[...], w1b_ref[...])
         + _dot(xd_ref[...], w1c_ref[...]) + b1_ref[...])
    t = jnp.maximum(t, 0.0)
    w = _dot(t, w2_ref[...]) + b2_ref[...]
    shm = _dot(sh_ref[...], wsh_ref[...])
    o_ref[...] = hs_ref[...] * shm * w


def _edge_summand(ea, xs, xd, sh, hs, w1, b1, w2_pad, b2_pad, wsh_pad,
                  block_rows):
    e = ea.shape[0]
    dp = w2_pad.shape[1]
    dmid = w1.shape[1]
    sh_dim = sh.shape[1]
    return pl.pallas_call(
        _edge_body,
        grid=(e // block_rows,),
        in_specs=[
            pl.BlockSpec((block_rows, NS), lambda i: (i, 0)),
            pl.BlockSpec((block_rows, NS), lambda i: (i, 0)),
            pl.BlockSpec((block_rows, NS), lambda i: (i, 0)),
            pl.BlockSpec((block_rows, sh_dim), lambda i: (i, 0)),
            pl.BlockSpec((block_rows, dp), lambda i: (i, 0)),
            pl.BlockSpec((NS, dmid), lambda i: (0, 0)),
            pl.BlockSpec((NS, dmid), lambda i: (0, 0)),
            pl.BlockSpec((NS, dmid), lambda i: (0, 0)),
            pl.BlockSpec((1, dmid), lambda i: (0, 0)),
            pl.BlockSpec((dmid, dp), lambda i: (0, 0)),
            pl.BlockSpec((1, dp), lambda i: (0, 0)),
            pl.BlockSpec((sh_dim, dp), lambda i: (0, 0)),
        ],
        out_specs=pl.BlockSpec((block_rows, dp), lambda i: (i, 0)),
        out_shape=jax.ShapeDtypeStruct((e, dp), F32),
    )(ea, xs, xd, sh, hs, w1[:NS], w1[NS:2 * NS], w1[2 * NS:],
      b1.reshape(1, -1), w2_pad, b2_pad.reshape(1, -1), wsh_pad)


def _post1_body(osum_ref, cnt_ref, x_ref, out_ref, s_ref, q_ref, *, din, dout):
    i = pl.program_id(0)
    xb = x_ref[...]
    if dout >= din:
        pad = jnp.zeros((xb.shape[0], dout - din), F32)
        res = jnp.concatenate([xb, pad], axis=1)
    else:
        res = xb[:, :dout]
    out = osum_ref[:, :dout] / jnp.maximum(cnt_ref[...], 1.0) + res
    out_ref[...] = out

    @pl.when(i == 0)
    def _():
        s_ref[...] = jnp.zeros_like(s_ref)
        q_ref[...] = jnp.zeros_like(q_ref)

    s_ref[...] += jnp.sum(out, axis=0, keepdims=True)
    q_ref[...] += jnp.sum(out * out, axis=0, keepdims=True)


def _post1(out_sum_pad, cnt, x, dout, block_rows):
    n, din = x.shape
    dp = out_sum_pad.shape[1]
    body = functools.partial(_post1_body, din=din, dout=dout)
    return pl.pallas_call(
        body,
        grid=(n // block_rows,),
        in_specs=[
            pl.BlockSpec((block_rows, dp), lambda i: (i, 0)),
            pl.BlockSpec((block_rows, 1), lambda i: (i, 0)),
            pl.BlockSpec((block_rows, din), lambda i: (i, 0)),
        ],
        out_specs=[
            pl.BlockSpec((block_rows, dout), lambda i: (i, 0)),
            pl.BlockSpec((1, dout), lambda i: (0, 0)),
            pl.BlockSpec((1, dout), lambda i: (0, 0)),
        ],
        out_shape=[
            jax.ShapeDtypeStruct((n, dout), F32),
            jax.ShapeDtypeStruct((1, dout), F32),
            jax.ShapeDtypeStruct((1, dout), F32),
        ],
    )(out_sum_pad[:, :dp], cnt, x)


def _post2_body(out_ref, s_ref, q_ref, x_ref, x32_ref):
    mean = s_ref[...] / N_NODES
    var = q_ref[...] / N_NODES - mean * mean
    xn = (out_ref[...] - mean) * jax.lax.rsqrt(var + 1e-5)
    x_ref[...] = xn
    x32_ref[...] = xn[:, :NS]


def _post2(out, s, q, block_rows):
    n, dout = out.shape
    return pl.pallas_call(
        _post2_body,
        grid=(n // block_rows,),
        in_specs=[
            pl.BlockSpec((block_rows, dout), lambda i: (i, 0)),
            pl.BlockSpec((1, dout), lambda i: (0, 0)),
            pl.BlockSpec((1, dout), lambda i: (0, 0)),
        ],
        out_specs=[
            pl.BlockSpec((block_rows, dout), lambda i: (i, 0)),
            pl.BlockSpec((block_rows, NS), lambda i: (i, 0)),
        ],
        out_shape=[
            jax.ShapeDtypeStruct((n, dout), F32),
            jax.ShapeDtypeStruct((n, NS), F32),
        ],
    )(out, s, q)


def _gather_rows(table, idx):
    return table[idx]


def _scatter_add(rows, dst, n):
    return jax.ops.segment_sum(rows, dst, num_segments=n)


def _degree_counts(dst, n):
    return jax.ops.segment_sum(
        jnp.ones((dst.shape[0], 1), F32), dst, num_segments=n)


def _pad_cols(a, dp):
    d = a.shape[1]
    if d == dp:
        return a
    return jnp.pad(a, ((0, 0), (0, dp - d)))


def _round16(d):
    return (d + 15) // 16 * 16


def kernel(node_attr, edge_index, edge_attr, edge_sh, node_mlp, edge_mlp,
           layers):
    src = edge_index[0]
    dst = edge_index[1]
    idx_flat = edge_index.reshape(-1)

    x = _mlp2_rows(node_attr, node_mlp, 1000)
    ea = _mlp2_rows(edge_attr, edge_mlp, 2000)
    cnt = _degree_counts(dst, N_NODES)

    x32 = x
    for layer in layers:
        din = layer['Wlin'].shape[0]
        dout = layer['Wlin'].shape[1]
        dp = _round16(dout)

        wlin_pad = _pad_cols(layer['Wlin'], dp)
        w2_pad = _pad_cols(layer['fcW2'], dp)
        b2_pad = jnp.pad(layer['fcb2'], (0, dp - dout))
        wsh_pad = _pad_cols(layer['Wsh'], dp)

        h = _node_linear(x, wlin_pad, 1000)
        xsd = _gather_rows(x32, idx_flat)
        xs, xd = xsd[:N_EDGES], xsd[N_EDGES:]
        hs = _gather_rows(h, src)
        summand = _edge_summand(
            ea, xs, xd, edge_sh, hs,
            layer['fcW1'], layer['fcb1'], w2_pad, b2_pad, wsh_pad, 2000)
        out_sum = _scatter_add(summand, dst, N_NODES)
        out, s, q = _post1(out_sum, cnt, x, dout, 1000)
        x, x32 = _post2(out, s, q, 1000)

    return (x, edge_index)


# speedup vs baseline: 49.2629x; 1.8323x over previous
import functools

import jax
import jax.numpy as jnp
from jax import lax
from jax.experimental import pallas as pl
from jax.experimental.pallas import tpu as pltpu
from jax.experimental.pallas import tpu_sc as plsc

N_NODES = 10000
N_EDGES = 160000
NS = 32

F32 = jnp.float32


def _dot(a, b):
    return jnp.dot(a, b, preferred_element_type=F32)


def _mlp2_body(x_ref, w1_ref, b1_ref, w2_ref, b2_ref, o_ref, *, dout):
    t = jnp.maximum(_dot(x_ref[...], w1_ref[...]) + b1_ref[...], 0.0)
    y = _dot(t, w2_ref[...]) + b2_ref[...]
    opad = o_ref.shape[1]
    if opad > dout:
        y = jnp.concatenate(
            [y, jnp.zeros((y.shape[0], opad - dout), F32)], axis=1)
    o_ref[...] = y


def _mlp2_rows(x, params, block_rows, out_pad=None):
    W1, b1, W2, b2 = params
    n, din = x.shape
    dmid, dout = W1.shape[1], W2.shape[1]
    opad = dout if out_pad is None else out_pad
    grid = (n // block_rows,)
    return pl.pallas_call(
        functools.partial(_mlp2_body, dout=dout),
        grid=grid,
        in_specs=[
            pl.BlockSpec((block_rows, din), lambda i: (i, 0)),
            pl.BlockSpec((din, dmid), lambda i: (0, 0)),
            pl.BlockSpec((1, dmid), lambda i: (0, 0)),
            pl.BlockSpec((dmid, dout), lambda i: (0, 0)),
            pl.BlockSpec((1, dout), lambda i: (0, 0)),
        ],
        out_specs=pl.BlockSpec((block_rows, opad), lambda i: (i, 0)),
        out_shape=jax.ShapeDtypeStruct((n, opad), F32),
    )(x, W1, b1.reshape(1, -1), W2, b2.reshape(1, -1))


def _matmul_body(x_ref, w_ref, o_ref):
    h = _dot(x_ref[...], w_ref[...])
    br = h.shape[0]
    x32 = x_ref[:, :NS]
    o_ref[...] = jnp.concatenate(
        [h, x32, jnp.zeros((br, 128 - NS), F32)], axis=1)


def _node_linear_ext(x, w_pad, block_rows):
    n, din = x.shape
    dp = w_pad.shape[1]
    return pl.pallas_call(
        _matmul_body,
        grid=(n // block_rows,),
        in_specs=[
            pl.BlockSpec((block_rows, din), lambda i: (i, 0)),
            pl.BlockSpec((din, dp), lambda i: (0, 0)),
        ],
        out_specs=pl.BlockSpec((block_rows, dp + 128), lambda i: (i, 0)),
        out_shape=jax.ShapeDtypeStruct((n, dp + 128), F32),
    )(x, w_pad)


def _edge_body(ea_ref, hse_ref, xd_ref, sh_ref,
               w1a_ref, w1b_ref, w1c_ref, b1_ref, w2_ref, b2_ref, wsh_ref,
               o_ref, *, n_chunks):
    dp = n_chunks * 128
    hs = hse_ref[:, :dp]
    xs = hse_ref[:, dp:dp + NS]
    xd = xd_ref[:, :NS]
    t = (_dot(ea_ref[...], w1a_ref[...]) + _dot(xs, w1b_ref[...])
         + _dot(xd, w1c_ref[...]) + b1_ref[...])
    t = jnp.maximum(t, 0.0)
    w = _dot(t, w2_ref[...]) + b2_ref[...]
    shm = _dot(sh_ref[...], wsh_ref[...])
    summ = hs * shm * w
    for c in range(n_chunks):
        o_ref[c] = summ[:, c * 128:(c + 1) * 128]


def _edge_summand(ea, hse, xd128, sh, w1, b1, w2_pad, b2_pad, wsh_pad,
                  block_rows):
    e = ea.shape[0]
    dp = w2_pad.shape[1]
    n_chunks = dp // 128
    dmid = w1.shape[1]
    sh_dim = sh.shape[1]
    return pl.pallas_call(
        functools.partial(_edge_body, n_chunks=n_chunks),
        grid=(e // block_rows,),
        in_specs=[
            pl.BlockSpec((block_rows, NS), lambda i: (i, 0)),
            pl.BlockSpec((block_rows, dp + 128), lambda i: (i, 0)),
            pl.BlockSpec((block_rows, 128), lambda i: (i, 0)),
            pl.BlockSpec((block_rows, sh_dim), lambda i: (i, 0)),
            pl.BlockSpec((NS, dmid), lambda i: (0, 0)),
            pl.BlockSpec((NS, dmid), lambda i: (0, 0)),
            pl.BlockSpec((NS, dmid), lambda i: (0, 0)),
            pl.BlockSpec((1, dmid), lambda i: (0, 0)),
            pl.BlockSpec((dmid, dp), lambda i: (0, 0)),
            pl.BlockSpec((1, dp), lambda i: (0, 0)),
            pl.BlockSpec((sh_dim, dp), lambda i: (0, 0)),
        ],
        out_specs=pl.BlockSpec((n_chunks, block_rows, 128),
                               lambda i: (0, i, 0)),
        out_shape=jax.ShapeDtypeStruct((n_chunks, e, 128), F32),
    )(ea, hse, xd128, sh, w1[:NS], w1[NS:2 * NS], w1[2 * NS:],
      b1.reshape(1, -1), w2_pad, b2_pad.reshape(1, -1), wsh_pad)


def _post1_body(osum_ref, cnt_ref, x_ref, out_ref, s_ref, q_ref, *, din, dout):
    i = pl.program_id(0)
    xb = x_ref[...]
    if dout >= din:
        pad = jnp.zeros((xb.shape[0], dout - din), F32)
        res = jnp.concatenate([xb, pad], axis=1)
    else:
        res = xb[:, :dout]
    out = osum_ref[:, :dout] / jnp.maximum(cnt_ref[:, :1], 1.0) + res
    out_ref[...] = out

    @pl.when(i == 0)
    def _():
        s_ref[...] = jnp.zeros_like(s_ref)
        q_ref[...] = jnp.zeros_like(q_ref)

    s_ref[...] += jnp.sum(out, axis=0, keepdims=True)
    q_ref[...] += jnp.sum(out * out, axis=0, keepdims=True)


def _post1(out_sum_pad, cnt, x, dout, block_rows):
    n, din = x.shape
    dp = out_sum_pad.shape[1]
    body = functools.partial(_post1_body, din=din, dout=dout)
    return pl.pallas_call(
        body,
        grid=(n // block_rows,),
        in_specs=[
            pl.BlockSpec((block_rows, dp), lambda i: (i, 0)),
            pl.BlockSpec((block_rows, 128), lambda i: (i, 0)),
            pl.BlockSpec((block_rows, din), lambda i: (i, 0)),
        ],
        out_specs=[
            pl.BlockSpec((block_rows, dout), lambda i: (i, 0)),
            pl.BlockSpec((1, dout), lambda i: (0, 0)),
            pl.BlockSpec((1, dout), lambda i: (0, 0)),
        ],
        out_shape=[
            jax.ShapeDtypeStruct((n, dout), F32),
            jax.ShapeDtypeStruct((1, dout), F32),
            jax.ShapeDtypeStruct((1, dout), F32),
        ],
    )(out_sum_pad[:, :dp], cnt, x)


def _post2_body(out_ref, s_ref, q_ref, x_ref, x128_ref):
    mean = s_ref[...] / N_NODES
    var = q_ref[...] / N_NODES - mean * mean
    xn = (out_ref[...] - mean) * jax.lax.rsqrt(var + 1e-5)
    x_ref[...] = xn
    x128_ref[...] = jnp.concatenate(
        [xn[:, :NS], jnp.zeros((xn.shape[0], 128 - NS), F32)], axis=1)


def _post2(out, s, q, block_rows):
    n, dout = out.shape
    return pl.pallas_call(
        _post2_body,
        grid=(n // block_rows,),
        in_specs=[
            pl.BlockSpec((block_rows, dout), lambda i: (i, 0)),
            pl.BlockSpec((1, dout), lambda i: (0, 0)),
            pl.BlockSpec((1, dout), lambda i: (0, 0)),
        ],
        out_specs=[
            pl.BlockSpec((block_rows, dout), lambda i: (i, 0)),
            pl.BlockSpec((block_rows, 128), lambda i: (i, 0)),
        ],
        out_shape=[
            jax.ShapeDtypeStruct((n, dout), F32),
            jax.ShapeDtypeStruct((n, 128), F32),
        ],
    )(out, s, q)


_SC_MESH = plsc.VectorSubcoreMesh(core_axis_name="c", subcore_axis_name="s")
_NW = 32
_NSUB = 16
_GCH = 128
_SCH = 80
N_PAD = 10240


def _sc_gather(table, idx3):
    nblk = idx3.shape[0]
    d = table.shape[1]
    nloops = -(-nblk // _NW)

    @functools.partial(
        pl.kernel, mesh=_SC_MESH,
        out_type=jax.ShapeDtypeStruct((nblk, _GCH, d), F32),
        scratch_types=[
            pltpu.VMEM((1, _GCH), jnp.int32),
            pltpu.VMEM((_GCH, d), F32),
        ],
    )
    def k(table_hbm, idx_hbm, out_hbm, idx_v, rows_v):
        wid = lax.axis_index("s") * 2 + lax.axis_index("c")

        @pl.loop(0, nloops)
        def _(i):
            g = i * _NW + wid

            @pl.when(g < nblk)
            def _():
                pltpu.sync_copy(idx_hbm.at[g], idx_v)
                pltpu.sync_copy(table_hbm.at[idx_v.at[0]], rows_v)
                pltpu.sync_copy(rows_v, out_hbm.at[g])

    return k(table, idx3)


def _sc_scatter_add(rows3, dst3, zeros):
    n_chunks, e, _ = rows3.shape
    dp = n_chunks * 128
    e_per_sub = e // _NSUB
    rps = N_PAD // _NSUB

    @functools.partial(
        pl.kernel, mesh=_SC_MESH,
        out_type=jax.ShapeDtypeStruct((N_PAD, dp), F32),
        scratch_types=[
            pltpu.VMEM((1, _SCH), jnp.int32),
            pltpu.VMEM((_SCH, 128), F32),
            pltpu.VMEM_SHARED((N_PAD, 128), F32),
            pltpu.SemaphoreType.DMA,
        ],
    )
    def k(rows_hbm, dst_hbm, zeros_hbm, out_hbm, idx_v, rv, acc, sem):
        cid = lax.axis_index("c")
        sid = lax.axis_index("s")
        r0 = sid * rps
        for i in range((n_chunks + 1) // 2):
            ch = 2 * i + cid

            @pl.when(ch < n_chunks)
            def _():
                pltpu.sync_copy(zeros_hbm, acc.at[pl.ds(r0, rps)])
                plsc.subcore_barrier()

                @pl.loop(0, e_per_sub // _SCH)
                def _(j):
                    g = sid * (e_per_sub // _SCH) + j
                    pltpu.sync_copy(dst_hbm.at[g], idx_v)
                    pltpu.sync_copy(
                        rows_hbm.at[ch].at[pl.ds(g * _SCH, _SCH)], rv)
                    pltpu.sync_copy(rv, acc.at[idx_v.at[0]], add=True)

                plsc.subcore_barrier()
                pltpu.sync_copy(
                    acc.at[pl.ds(r0, rps)],
                    out_hbm.at[pl.ds(r0, rps), pl.ds(ch * 128, 128)])
                plsc.subcore_barrier()

    return k(rows3, dst3, zeros)


def _sc_degree_counts(dst3, ones, zeros):
    e = dst3.shape[0] * _SCH
    e_per_sub = e // _NSUB
    rps = N_PAD // _NSUB

    @functools.partial(
        pl.kernel, mesh=_SC_MESH,
        out_type=jax.ShapeDtypeStruct((N_PAD, 128), F32),
        scratch_types=[
            pltpu.VMEM((1, _SCH), jnp.int32),
            pltpu.VMEM((_SCH, 128), F32),
            pltpu.VMEM_SHARED((N_PAD, 128), F32),
        ],
    )
    def k(dst_hbm, ones_hbm, zeros_hbm, out_hbm, idx_v, ones_v, acc):
        cid = lax.axis_index("c")
        sid = lax.axis_index("s")
        r0 = sid * rps

        @pl.when(cid == 0)
        def _():
            pltpu.sync_copy(ones_hbm, ones_v)
            pltpu.sync_copy(zeros_hbm, acc.at[pl.ds(r0, rps)])
            plsc.subcore_barrier()

            @pl.loop(0, e_per_sub // _SCH)
            def _(j):
                g = sid * (e_per_sub // _SCH) + j
                pltpu.sync_copy(dst_hbm.at[g], idx_v)
                pltpu.sync_copy(ones_v, acc.at[idx_v.at[0]], add=True)

            plsc.subcore_barrier()
            pltpu.sync_copy(acc.at[pl.ds(r0, rps)],
                            out_hbm.at[pl.ds(r0, rps)])

    return k(dst3, ones, zeros)


def _pad_cols(a, dp):
    d = a.shape[1]
    if d == dp:
        return a
    return jnp.pad(a, ((0, 0), (0, dp - d)))


def kernel(node_attr, edge_index, edge_attr, edge_sh, node_mlp, edge_mlp,
           layers):
    src = edge_index[0]
    dst = edge_index[1]
    src3 = src.reshape(N_EDGES // _GCH, 1, _GCH)
    dstg3 = dst.reshape(N_EDGES // _GCH, 1, _GCH)
    dst3 = dst.reshape(N_EDGES // _SCH, 1, _SCH)
    zeros = jnp.zeros((N_PAD // _NSUB, 128), F32)
    ones0 = jnp.zeros((_SCH, 128), F32).at[:, 0].set(1.0)

    x128 = _mlp2_rows(node_attr, node_mlp, 1000, out_pad=128)
    x = x128[:, :NS]
    ea = _mlp2_rows(edge_attr, edge_mlp, 2000)
    cnt = _sc_degree_counts(dst3, ones0, zeros)

    for layer in layers:
        dout = layer['Wlin'].shape[1]
        n_chunks = -(-dout // 128)
        dp = 128 * n_chunks

        wlin_pad = _pad_cols(layer['Wlin'], dp)
        w2_pad = _pad_cols(layer['fcW2'], dp)
        b2_pad = jnp.pad(layer['fcb2'], (0, dp - dout))
        wsh_pad = _pad_cols(layer['Wsh'], dp)

        ht = _node_linear_ext(x, wlin_pad, 1000)
        hse = _sc_gather(ht, src3).reshape(N_EDGES, dp + 128)
        xd128 = _sc_gather(x128, dstg3).reshape(N_EDGES, 128)
        summand = _edge_summand(
            ea, hse, xd128, edge_sh,
            layer['fcW1'], layer['fcb1'], w2_pad, b2_pad, wsh_pad, 2000)
        out_sum = _sc_scatter_add(summand, dst3, zeros)
        out, s, q = _post1(out_sum, cnt, x, dout, 1000)
        x, x128 = _post2(out, s, q, 1000)

    return (x, edge_index)


# speedup vs baseline: 51.3861x; 1.0431x over previous
import functools

import jax
import jax.numpy as jnp
from jax import lax
from jax.experimental import pallas as pl
from jax.experimental.pallas import tpu as pltpu
from jax.experimental.pallas import tpu_sc as plsc

N_NODES = 10000
N_EDGES = 160000
NS = 32

F32 = jnp.float32


def _dot(a, b):
    return jnp.dot(a, b, preferred_element_type=F32)


def _mlp2_body(x_ref, w1_ref, b1_ref, w2_ref, b2_ref, o_ref, *, dout):
    t = jnp.maximum(_dot(x_ref[...], w1_ref[...]) + b1_ref[...], 0.0)
    y = _dot(t, w2_ref[...]) + b2_ref[...]
    opad = o_ref.shape[1]
    if opad > dout:
        y = jnp.concatenate(
            [y, jnp.zeros((y.shape[0], opad - dout), F32)], axis=1)
    o_ref[...] = y


def _mlp2_rows(x, params, block_rows, out_pad=None):
    W1, b1, W2, b2 = params
    n, din = x.shape
    dmid, dout = W1.shape[1], W2.shape[1]
    opad = dout if out_pad is None else out_pad
    grid = (n // block_rows,)
    return pl.pallas_call(
        functools.partial(_mlp2_body, dout=dout),
        grid=grid,
        in_specs=[
            pl.BlockSpec((block_rows, din), lambda i: (i, 0)),
            pl.BlockSpec((din, dmid), lambda i: (0, 0)),
            pl.BlockSpec((1, dmid), lambda i: (0, 0)),
            pl.BlockSpec((dmid, dout), lambda i: (0, 0)),
            pl.BlockSpec((1, dout), lambda i: (0, 0)),
        ],
        out_specs=pl.BlockSpec((block_rows, opad), lambda i: (i, 0)),
        out_shape=jax.ShapeDtypeStruct((n, opad), F32),
    )(x, W1, b1.reshape(1, -1), W2, b2.reshape(1, -1))


def _matmul_body(x_ref, w_ref, o_ref):
    h = _dot(x_ref[...], w_ref[...])
    br = h.shape[0]
    x32 = x_ref[:, :NS]
    o_ref[...] = jnp.concatenate(
        [h, x32, jnp.zeros((br, 128 - NS), F32)], axis=1)


def _node_linear_ext(x, w_pad, block_rows):
    n, din = x.shape
    dp = w_pad.shape[1]
    return pl.pallas_call(
        _matmul_body,
        grid=(n // block_rows,),
        in_specs=[
            pl.BlockSpec((block_rows, din), lambda i: (i, 0)),
            pl.BlockSpec((din, dp), lambda i: (0, 0)),
        ],
        out_specs=pl.BlockSpec((block_rows, dp + 128), lambda i: (i, 0)),
        out_shape=jax.ShapeDtypeStruct((n, dp + 128), F32),
    )(x, w_pad)


def _edge_body(ea_ref, hse_ref, xd_ref, sh_ref,
               w1a_ref, w1b_ref, w1c_ref, b1_ref, w2_ref, b2_ref, wsh_ref,
               o_ref, *, n_chunks):
    dp = n_chunks * 128
    hs = hse_ref[:, :dp]
    xs = hse_ref[:, dp:dp + NS]
    xd = xd_ref[:, :NS]
    t = (_dot(ea_ref[...], w1a_ref[...]) + _dot(xs, w1b_ref[...])
         + _dot(xd, w1c_ref[...]) + b1_ref[...])
    t = jnp.maximum(t, 0.0)
    w = _dot(t, w2_ref[...]) + b2_ref[...]
    shm = _dot(sh_ref[...], wsh_ref[...])
    summ = hs * shm * w
    for c in range(n_chunks):
        o_ref[c] = summ[:, c * 128:(c + 1) * 128]


def _edge_summand(ea, hse, xd128, sh, w1, b1, w2_pad, b2_pad, wsh_pad,
                  block_rows):
    e = ea.shape[0]
    dp = w2_pad.shape[1]
    n_chunks = dp // 128
    dmid = w1.shape[1]
    sh_dim = sh.shape[1]
    return pl.pallas_call(
        functools.partial(_edge_body, n_chunks=n_chunks),
        grid=(e // block_rows,),
        in_specs=[
            pl.BlockSpec((block_rows, NS), lambda i: (i, 0)),
            pl.BlockSpec((block_rows, dp + 128), lambda i: (i, 0)),
            pl.BlockSpec((block_rows, 128), lambda i: (i, 0)),
            pl.BlockSpec((block_rows, sh_dim), lambda i: (i, 0)),
            pl.BlockSpec((NS, dmid), lambda i: (0, 0)),
            pl.BlockSpec((NS, dmid), lambda i: (0, 0)),
            pl.BlockSpec((NS, dmid), lambda i: (0, 0)),
            pl.BlockSpec((1, dmid), lambda i: (0, 0)),
            pl.BlockSpec((dmid, dp), lambda i: (0, 0)),
            pl.BlockSpec((1, dp), lambda i: (0, 0)),
            pl.BlockSpec((sh_dim, dp), lambda i: (0, 0)),
        ],
        out_specs=pl.BlockSpec((n_chunks, block_rows, 128),
                               lambda i: (0, i, 0)),
        out_shape=jax.ShapeDtypeStruct((n_chunks, e, 128), F32),
    )(ea, hse, xd128, sh, w1[:NS], w1[NS:2 * NS], w1[2 * NS:],
      b1.reshape(1, -1), w2_pad, b2_pad.reshape(1, -1), wsh_pad)


def _post1_body(osum_ref, cnt_ref, x_ref, out_ref, s_ref, q_ref, *, din, dout):
    i = pl.program_id(0)
    xb = x_ref[...]
    if dout >= din:
        pad = jnp.zeros((xb.shape[0], dout - din), F32)
        res = jnp.concatenate([xb, pad], axis=1)
    else:
        res = xb[:, :dout]
    out = osum_ref[:, :dout] / jnp.maximum(cnt_ref[:, :1], 1.0) + res
    out_ref[...] = out

    @pl.when(i == 0)
    def _():
        s_ref[...] = jnp.zeros_like(s_ref)
        q_ref[...] = jnp.zeros_like(q_ref)

    s_ref[...] += jnp.sum(out, axis=0, keepdims=True)
    q_ref[...] += jnp.sum(out * out, axis=0, keepdims=True)


def _post1(out_sum_pad, cnt, x, dout, block_rows):
    n, din = x.shape
    dp = out_sum_pad.shape[1]
    body = functools.partial(_post1_body, din=din, dout=dout)
    return pl.pallas_call(
        body,
        grid=(n // block_rows,),
        in_specs=[
            pl.BlockSpec((block_rows, dp), lambda i: (i, 0)),
            pl.BlockSpec((block_rows, 128), lambda i: (i, 0)),
            pl.BlockSpec((block_rows, din), lambda i: (i, 0)),
        ],
        out_specs=[
            pl.BlockSpec((block_rows, dout), lambda i: (i, 0)),
            pl.BlockSpec((1, dout), lambda i: (0, 0)),
            pl.BlockSpec((1, dout), lambda i: (0, 0)),
        ],
        out_shape=[
            jax.ShapeDtypeStruct((n, dout), F32),
            jax.ShapeDtypeStruct((1, dout), F32),
            jax.ShapeDtypeStruct((1, dout), F32),
        ],
    )(out_sum_pad[:, :dp], cnt, x)


def _post2_body(out_ref, s_ref, q_ref, x_ref, x128_ref):
    mean = s_ref[...] / N_NODES
    var = q_ref[...] / N_NODES - mean * mean
    xn = (out_ref[...] - mean) * jax.lax.rsqrt(var + 1e-5)
    x_ref[...] = xn
    x128_ref[...] = jnp.concatenate(
        [xn[:, :NS], jnp.zeros((xn.shape[0], 128 - NS), F32)], axis=1)


def _post2(out, s, q, block_rows):
    n, dout = out.shape
    return pl.pallas_call(
        _post2_body,
        grid=(n // block_rows,),
        in_specs=[
            pl.BlockSpec((block_rows, dout), lambda i: (i, 0)),
            pl.BlockSpec((1, dout), lambda i: (0, 0)),
            pl.BlockSpec((1, dout), lambda i: (0, 0)),
        ],
        out_specs=[
            pl.BlockSpec((block_rows, dout), lambda i: (i, 0)),
            pl.BlockSpec((block_rows, 128), lambda i: (i, 0)),
        ],
        out_shape=[
            jax.ShapeDtypeStruct((n, dout), F32),
            jax.ShapeDtypeStruct((n, 128), F32),
        ],
    )(out, s, q)


_SC_MESH = plsc.VectorSubcoreMesh(core_axis_name="c", subcore_axis_name="s")
_NW = 32
_NSUB = 16
_GCH = 128
_SCH = 80
N_PAD = 10240


def _sc_gather(table, idx3):
    nblk = idx3.shape[0]
    gch = idx3.shape[2]
    d = table.shape[1]
    npair = -(-nblk // (2 * _NW))

    @functools.partial(
        pl.kernel, mesh=_SC_MESH,
        out_type=jax.ShapeDtypeStruct((nblk, gch, d), F32),
        scratch_types=[
            pltpu.VMEM((1, gch), jnp.int32),
            pltpu.VMEM((1, gch), jnp.int32),
            pltpu.VMEM((gch, d), F32),
            pltpu.VMEM((gch, d), F32),
            pltpu.SemaphoreType.DMA,
            pltpu.SemaphoreType.DMA,
            pltpu.SemaphoreType.DMA,
            pltpu.SemaphoreType.DMA,
        ],
    )
    def k(table_hbm, idx_hbm, out_hbm, iv0, iv1, rv0, rv1, sg0, sg1,
          so0, so1):
        wid = lax.axis_index("s") * 2 + lax.axis_index("c")

        @pl.loop(0, npair)
        def _(j):
            g0 = (2 * j) * _NW + wid
            g1 = (2 * j + 1) * _NW + wid

            @pl.when(g0 < nblk)
            def _():
                @pl.when(j > 0)
                def _():
                    pltpu.make_async_copy(rv0, out_hbm.at[g0], so0).wait()
                pltpu.sync_copy(idx_hbm.at[g0], iv0)
                pltpu.async_copy(table_hbm.at[iv0.at[0]], rv0, sg0)

            @pl.when(g1 < nblk)
            def _():
                @pl.when(j > 0)
                def _():
                    pltpu.make_async_copy(rv1, out_hbm.at[g1], so1).wait()
                pltpu.sync_copy(idx_hbm.at[g1], iv1)
                pltpu.async_copy(table_hbm.at[iv1.at[0]], rv1, sg1)

            @pl.when(g0 < nblk)
            def _():
                pltpu.make_async_copy(table_hbm.at[iv0.at[0]], rv0,
                                      sg0).wait()
                pltpu.async_copy(rv0, out_hbm.at[g0], so0)

            @pl.when(g1 < nblk)
            def _():
                pltpu.make_async_copy(table_hbm.at[iv1.at[0]], rv1,
                                      sg1).wait()
                pltpu.async_copy(rv1, out_hbm.at[g1], so1)

        pltpu.make_async_copy(rv0, out_hbm.at[wid], so0).wait()
        pltpu.make_async_copy(rv1, out_hbm.at[wid], so1).wait()

    return k(table, idx3)


def _sc_scatter_add(rows3, dst3, zeros):
    n_chunks, e, _ = rows3.shape
    dp = n_chunks * 128
    e_per_sub = e // _NSUB
    rps = N_PAD // _NSUB

    @functools.partial(
        pl.kernel, mesh=_SC_MESH,
        out_type=jax.ShapeDtypeStruct((N_PAD, dp), F32),
        scratch_types=[
            pltpu.VMEM((1, _SCH), jnp.int32),
            pltpu.VMEM((_SCH, 128), F32),
            pltpu.VMEM_SHARED((N_PAD, 128), F32),
            pltpu.SemaphoreType.DMA,
        ],
    )
    def k(rows_hbm, dst_hbm, zeros_hbm, out_hbm, idx_v, rv, acc, sem):
        cid = lax.axis_index("c")
        sid = lax.axis_index("s")
        r0 = sid * rps
        for i in range((n_chunks + 1) // 2):
            ch = 2 * i + cid

            @pl.when(ch < n_chunks)
            def _():
                pltpu.sync_copy(zeros_hbm, acc.at[pl.ds(r0, rps)])
                plsc.subcore_barrier()

                @pl.loop(0, e_per_sub // _SCH)
                def _(j):
                    g = sid * (e_per_sub // _SCH) + j
                    pltpu.sync_copy(dst_hbm.at[g], idx_v)
                    pltpu.sync_copy(
                        rows_hbm.at[ch].at[pl.ds(g * _SCH, _SCH)], rv)
                    pltpu.sync_copy(rv, acc.at[idx_v.at[0]], add=True)

                plsc.subcore_barrier()
                pltpu.sync_copy(
                    acc.at[pl.ds(r0, rps)],
                    out_hbm.at[pl.ds(r0, rps), pl.ds(ch * 128, 128)])
                plsc.subcore_barrier()

    return k(rows3, dst3, zeros)


def _sc_degree_counts(dst3, ones, zeros):
    e = dst3.shape[0] * _SCH
    e_per_sub = e // _NSUB
    rps = N_PAD // _NSUB

    @functools.partial(
        pl.kernel, mesh=_SC_MESH,
        out_type=jax.ShapeDtypeStruct((N_PAD, 128), F32),
        scratch_types=[
            pltpu.VMEM((1, _SCH), jnp.int32),
            pltpu.VMEM((_SCH, 128), F32),
            pltpu.VMEM_SHARED((N_PAD, 128), F32),
        ],
    )
    def k(dst_hbm, ones_hbm, zeros_hbm, out_hbm, idx_v, ones_v, acc):
        cid = lax.axis_index("c")
        sid = lax.axis_index("s")
        r0 = sid * rps

        @pl.when(cid == 0)
        def _():
            pltpu.sync_copy(ones_hbm, ones_v)
            pltpu.sync_copy(zeros_hbm, acc.at[pl.ds(r0, rps)])
            plsc.subcore_barrier()

            @pl.loop(0, e_per_sub // _SCH)
            def _(j):
                g = sid * (e_per_sub // _SCH) + j
                pltpu.sync_copy(dst_hbm.at[g], idx_v)
                pltpu.sync_copy(ones_v, acc.at[idx_v.at[0]], add=True)

            plsc.subcore_barrier()
            pltpu.sync_copy(acc.at[pl.ds(r0, rps)],
                            out_hbm.at[pl.ds(r0, rps)])

    return k(dst3, ones, zeros)


def _pad_cols(a, dp):
    d = a.shape[1]
    if d == dp:
        return a
    return jnp.pad(a, ((0, 0), (0, dp - d)))


def kernel(node_attr, edge_index, edge_attr, edge_sh, node_mlp, edge_mlp,
           layers):
    src = edge_index[0]
    dst = edge_index[1]
    src3 = src.reshape(N_EDGES // 64, 1, 64)
    dstg3 = dst.reshape(N_EDGES // _GCH, 1, _GCH)
    dst3 = dst.reshape(N_EDGES // _SCH, 1, _SCH)
    zeros = jnp.zeros((N_PAD // _NSUB, 128), F32)
    ones0 = jnp.zeros((_SCH, 128), F32).at[:, 0].set(1.0)

    x128 = _mlp2_rows(node_attr, node_mlp, 1000, out_pad=128)
    x = x128[:, :NS]
    ea = _mlp2_rows(edge_attr, edge_mlp, 2000)
    cnt = _sc_degree_counts(dst3, ones0, zeros)

    for layer in layers:
        dout = layer['Wlin'].shape[1]
        n_chunks = -(-dout // 128)
        dp = 128 * n_chunks

        wlin_pad = _pad_cols(layer['Wlin'], dp)
        w2_pad = _pad_cols(layer['fcW2'], dp)
        b2_pad = jnp.pad(layer['fcb2'], (0, dp - dout))
        wsh_pad = _pad_cols(layer['Wsh'], dp)

        ht = _node_linear_ext(x, wlin_pad, 1000)
        hse = _sc_gather(ht, src3).reshape(N_EDGES, dp + 128)
        xd128 = _sc_gather(x128, dstg3).reshape(N_EDGES, 128)
        summand = _edge_summand(
            ea, hse, xd128, edge_sh,
            layer['fcW1'], layer['fcb1'], w2_pad, b2_pad, wsh_pad, 2000)
        out_sum = _sc_scatter_add(summand, dst3, zeros)
        out, s, q = _post1(out_sum, cnt, x, dout, 1000)
        x, x128 = _post2(out, s, q, 1000)

    return (x, edge_index)
